# baseline (speedup 1.0000x reference)
import functools

import jax
import jax.numpy as jnp
from jax import lax
from jax.experimental import pallas as pl
from jax.experimental.pallas import tpu as pltpu
from jax.experimental.pallas import tpu_sc as plsc

N = 10000
E = 320000
D = 128

NC = 2
NS = 16
NW = NC * NS

CH = 128
CPT = 80
CPP = 40
N0 = 160
N1 = 2 * CPT - N0
EPT = CH * CPT
E_PAD = EPT * NW
R = 10240
RPT = R // NS

_mesh = plsc.VectorSubcoreMesh(core_axis_name="c", subcore_axis_name="s")


DW = 128


@functools.partial(
    pl.kernel,
    out_type=jax.ShapeDtypeStruct((NC, R, DW), jnp.float32),
    mesh=_mesh,
    scratch_types=[
        pltpu.VMEM_SHARED((R, DW), jnp.float32),
        pltpu.VMEM((CPT, CH), jnp.int32),
        pltpu.VMEM((CH, DW), jnp.float32),
    ],
)
def _deg_kernel(dst_hbm, ones_hbm, z_hbm, out_hbm, accd, didx, ones_v):
    c = lax.axis_index("c")
    s = lax.axis_index("s")
    wid = c * NS + s
    pltpu.sync_copy(z_hbm, accd.at[pl.ds(s * RPT, RPT)])
    pltpu.sync_copy(ones_hbm, ones_v)
    pltpu.sync_copy(dst_hbm.at[pl.ds(wid * CPT, CPT)], didx)
    plsc.subcore_barrier()

    def body(j, carry):
        pltpu.sync_copy(ones_v, accd.at[didx.at[j]], add=True)
        return carry

    lax.fori_loop(0, CPT, body, 0)
    plsc.subcore_barrier()
    pltpu.sync_copy(accd.at[pl.ds(s * RPT, RPT)],
                    out_hbm.at[c, pl.ds(s * RPT, RPT)])


@functools.partial(
    pl.kernel,
    out_type=jax.ShapeDtypeStruct((NC, R, D), jnp.float32),
    mesh=_mesh,
    scratch_types=[
        pltpu.VMEM_SHARED((R, D), jnp.float32),
        pltpu.VMEM((CPP, CH), jnp.int32),
        pltpu.VMEM((CPP, CH), jnp.int32),
        pltpu.VMEM((CH, D), jnp.float32),
        pltpu.VMEM((CH, D), jnp.float32),
        pltpu.SemaphoreType.DMA,
        pltpu.SemaphoreType.DMA,
    ],
)
def _agg_kernel(g_hbm, src_hbm, dst_hbm, z_hbm, out_hbm,
                acc, sidx, didx, buf0, buf1, sem0, sem1):
    c = lax.axis_index("c")
    s = lax.axis_index("s")
    pltpu.sync_copy(z_hbm, acc.at[pl.ds(s * RPT, RPT)])
    plsc.subcore_barrier()

    def run_chunks(row0, n):
        done = 0
        while done < n:
            sz = min(CPP, n - done)
            base = row0 + done
            pltpu.sync_copy(src_hbm.at[pl.ds(base, sz)], sidx.at[pl.ds(0, sz)])
            pltpu.sync_copy(dst_hbm.at[pl.ds(base, sz)], didx.at[pl.ds(0, sz)])
            pltpu.async_copy(g_hbm.at[sidx.at[0]], buf0, sem0)

            def body(j, carry):
                pltpu.async_copy(g_hbm.at[sidx.at[j + 1]], buf1, sem1)
                pltpu.make_async_copy(g_hbm.at[sidx.at[j]], buf0, sem0).wait()
                pltpu.sync_copy(buf0, acc.at[didx.at[j]], add=True)

                @pl.when(j + 2 < sz)
                def _():
                    pltpu.async_copy(g_hbm.at[sidx.at[j + 2]], buf0, sem0)

                pltpu.make_async_copy(g_hbm.at[sidx.at[j + 1]], buf1, sem1).wait()
                pltpu.sync_copy(buf1, acc.at[didx.at[j + 1]], add=True)
                return carry

            lax.fori_loop(0, sz // 2, lambda i, cr: body(2 * i, cr), 0)
            done += sz

    if N0 > 0:
        @pl.when(c == 0)
        def _():
            run_chunks(s * N0, N0)
    if N1 > 0:
        @pl.when(c == 1)
        def _():
            run_chunks(NS * N0 + s * N1, N1)

    plsc.subcore_barrier()
    pltpu.sync_copy(acc.at[pl.ds(s * RPT, RPT)],
                    out_hbm.at[c, pl.ds(s * RPT, RPT)])


BM = 2000


def _mm1_body(x_ref, w_ref, da_ref, db_ref, g_ref, dv_ref):
    deg = da_ref[...] + db_ref[...] + 1.0
    dinv = lax.rsqrt(deg)
    h = jnp.dot(x_ref[...], w_ref[...], preferred_element_type=jnp.float32)
    g_ref[...] = h * dinv[:, :1]
    dv_ref[...] = dinv


_mm1 = pl.pallas_call(
    _mm1_body,
    grid=(N // BM,),
    in_specs=[
        pl.BlockSpec((BM, D), lambda i: (i, 0)),
        pl.BlockSpec((D, D), lambda i: (0, 0)),
        pl.BlockSpec((BM, 16), lambda i: (i, 0)),
        pl.BlockSpec((BM, 16), lambda i: (i, 0)),
    ],
    out_specs=[
        pl.BlockSpec((BM, D), lambda i: (i, 0)),
        pl.BlockSpec((BM, 16), lambda i: (i, 0)),
    ],
    out_shape=[
        jax.ShapeDtypeStruct((N, D), jnp.float32),
        jax.ShapeDtypeStruct((N, 16), jnp.float32),
    ],
)


def _mm2_body(aa_ref, ab_ref, g1_ref, dv_ref, b1_ref, w_ref, g2_ref):
    dv = dv_ref[...][:, :1]
    z = dv * (aa_ref[...] + ab_ref[...] + g1_ref[...]) + b1_ref[...]
    r = jnp.maximum(z, 0.0)
    h = jnp.dot(r, w_ref[...], preferred_element_type=jnp.float32)
    g2_ref[...] = h * dv


_mm2 = pl.pallas_call(
    _mm2_body,
    grid=(N // BM,),
    in_specs=[
        pl.BlockSpec((BM, D), lambda i: (i, 0)),
        pl.BlockSpec((BM, D), lambda i: (i, 0)),
        pl.BlockSpec((BM, D), lambda i: (i, 0)),
        pl.BlockSpec((BM, 16), lambda i: (i, 0)),
        pl.BlockSpec((1, D), lambda i: (0, 0)),
        pl.BlockSpec((D, D), lambda i: (0, 0)),
    ],
    out_specs=pl.BlockSpec((BM, D), lambda i: (i, 0)),
    out_shape=jax.ShapeDtypeStruct((N, D), jnp.float32),
)


def _final_body(aa_ref, ab_ref, g2_ref, dv_ref, b2_ref, o_ref):
    dv = dv_ref[...][:, :1]
    o_ref[...] = dv * (aa_ref[...] + ab_ref[...] + g2_ref[...]) + b2_ref[...]


_final = pl.pallas_call(
    _final_body,
    grid=(N // BM,),
    in_specs=[
        pl.BlockSpec((BM, D), lambda i: (i, 0)),
        pl.BlockSpec((BM, D), lambda i: (i, 0)),
        pl.BlockSpec((BM, D), lambda i: (i, 0)),
        pl.BlockSpec((BM, 16), lambda i: (i, 0)),
        pl.BlockSpec((1, D), lambda i: (0, 0)),
    ],
    out_specs=pl.BlockSpec((BM, D), lambda i: (i, 0)),
    out_shape=jax.ShapeDtypeStruct((N, D), jnp.float32),
)


def kernel(x, edge_index, W1, b1, W2, b2):
    src = edge_index[0]
    dst = edge_index[1]
    pad = E_PAD - E
    srcp = jnp.concatenate([src, jnp.zeros((pad,), jnp.int32)])
    dstp = jnp.concatenate([dst, jnp.full((pad,), N, jnp.int32)])
    src3 = srcp.reshape(NW * CPT, CH)
    dst3 = dstp.reshape(NW * CPT, CH)

    z128 = jnp.zeros((RPT, D), jnp.float32)
    zdw = jnp.zeros((RPT, DW), jnp.float32)
    onesdw = jnp.ones((CH, DW), jnp.float32)

    degp = _deg_kernel(dst3, onesdw, zdw)
    g1, dinv = _mm1(x, W1, degp[0, :N, :16], degp[1, :N, :16])
    agg1 = _agg_kernel(g1, src3, dst3, z128)
    g2 = _mm2(agg1[0, :N], agg1[1, :N], g1, dinv,
              b1.reshape(1, D), W2)
    agg2 = _agg_kernel(g2, src3, dst3, z128)
    out = _final(agg2[0, :N], agg2[1, :N], g2, dinv,
                 b2.reshape(1, D))
    return out


# speedup vs baseline: 1.2367x; 1.2367x over previous
import functools

import jax
import jax.numpy as jnp
from jax import lax
from jax.experimental import pallas as pl
from jax.experimental.pallas import tpu as pltpu
from jax.experimental.pallas import tpu_sc as plsc

N = 10000
E = 320000
D = 128

NC = 2
NS = 16
NW = NC * NS

CH = 128
CPT = 80
CPP = 40
N0 = 152
N1 = 2 * CPT - N0
EPT = CH * CPT
E_PAD = EPT * NW
R = 10240
RPT = R // NS

_mesh = plsc.VectorSubcoreMesh(core_axis_name="c", subcore_axis_name="s")


DW = 128


@functools.partial(
    pl.kernel,
    out_type=jax.ShapeDtypeStruct((NC, R, DW), jnp.float32),
    mesh=_mesh,
    scratch_types=[
        pltpu.VMEM_SHARED((R, DW), jnp.float32),
        pltpu.VMEM((CPT, CH), jnp.int32),
        pltpu.VMEM((CH, DW), jnp.float32),
    ],
)
def _deg_kernel(dst_hbm, ones_hbm, z_hbm, out_hbm, accd, didx, ones_v):
    c = lax.axis_index("c")
    s = lax.axis_index("s")
    wid = c * NS + s
    pltpu.sync_copy(z_hbm, accd.at[pl.ds(s * RPT, RPT)])
    pltpu.sync_copy(ones_hbm, ones_v)
    pltpu.sync_copy(dst_hbm.at[pl.ds(wid * CPT, CPT)], didx)
    plsc.subcore_barrier()

    def body(j, carry):
        pltpu.sync_copy(ones_v, accd.at[didx.at[j]], add=True)
        return carry

    lax.fori_loop(0, CPT, body, 0)
    plsc.subcore_barrier()
    pltpu.sync_copy(accd.at[pl.ds(s * RPT, RPT)],
                    out_hbm.at[c, pl.ds(s * RPT, RPT)])


@functools.partial(
    pl.kernel,
    out_type=jax.ShapeDtypeStruct((NC, R, D), jnp.float32),
    mesh=_mesh,
    scratch_types=[
        pltpu.VMEM_SHARED((R, D), jnp.float32),
        pltpu.VMEM((CPP, CH), jnp.int32),
        pltpu.VMEM((CPP, CH), jnp.int32),
        pltpu.VMEM((CH, D), jnp.float32),
        pltpu.VMEM((CH, D), jnp.float32),
        pltpu.SemaphoreType.DMA,
        pltpu.SemaphoreType.DMA,
    ],
)
def _agg_kernel(g_hbm, src_hbm, dst_hbm, z_hbm, out_hbm,
                acc, sidx, didx, buf0, buf1, sem0, sem1):
    c = lax.axis_index("c")
    s = lax.axis_index("s")
    pltpu.sync_copy(z_hbm, acc.at[pl.ds(s * RPT, RPT)])
    plsc.subcore_barrier()

    def run_chunks(row0, n):
        done = 0
        while done < n:
            sz = min(CPP, n - done)
            base = row0 + done
            pltpu.sync_copy(src_hbm.at[pl.ds(base, sz)], sidx.at[pl.ds(0, sz)])
            pltpu.sync_copy(dst_hbm.at[pl.ds(base, sz)], didx.at[pl.ds(0, sz)])
            pltpu.async_copy(g_hbm.at[sidx.at[0]], buf0, sem0)

            def body(j, carry):
                pltpu.async_copy(g_hbm.at[sidx.at[j + 1]], buf1, sem1)
                pltpu.make_async_copy(g_hbm.at[sidx.at[j]], buf0, sem0).wait()
                pltpu.sync_copy(buf0, acc.at[didx.at[j]], add=True)

                @pl.when(j + 2 < sz)
                def _():
                    pltpu.async_copy(g_hbm.at[sidx.at[j + 2]], buf0, sem0)

                pltpu.make_async_copy(g_hbm.at[sidx.at[j + 1]], buf1, sem1).wait()
                pltpu.sync_copy(buf1, acc.at[didx.at[j + 1]], add=True)
                return carry

            lax.fori_loop(0, sz // 2, lambda i, cr: body(2 * i, cr), 0)
            done += sz

    if N0 > 0:
        @pl.when(c == 0)
        def _():
            run_chunks(s * N0, N0)
    if N1 > 0:
        @pl.when(c == 1)
        def _():
            run_chunks(NS * N0 + s * N1, N1)

    plsc.subcore_barrier()
    pltpu.sync_copy(acc.at[pl.ds(s * RPT, RPT)],
                    out_hbm.at[c, pl.ds(s * RPT, RPT)])


BM = 2000


def _mm1_body(x_ref, w_ref, da_ref, db_ref, g_ref, dv_ref):
    deg = da_ref[...] + db_ref[...] + 1.0
    dinv = lax.rsqrt(deg)
    h = jnp.dot(x_ref[...], w_ref[...], preferred_element_type=jnp.float32)
    g_ref[...] = h * dinv[:, :1]
    dv_ref[...] = dinv


_mm1 = pl.pallas_call(
    _mm1_body,
    grid=(N // BM,),
    in_specs=[
        pl.BlockSpec((BM, D), lambda i: (i, 0)),
        pl.BlockSpec((D, D), lambda i: (0, 0)),
        pl.BlockSpec((BM, 16), lambda i: (i, 0)),
        pl.BlockSpec((BM, 16), lambda i: (i, 0)),
    ],
    out_specs=[
        pl.BlockSpec((BM, D), lambda i: (i, 0)),
        pl.BlockSpec((BM, 16), lambda i: (i, 0)),
    ],
    out_shape=[
        jax.ShapeDtypeStruct((N, D), jnp.float32),
        jax.ShapeDtypeStruct((N, 16), jnp.float32),
    ],
)


def _mm2_body(aa_ref, ab_ref, g1_ref, dv_ref, b1_ref, w_ref, g2_ref):
    dv = dv_ref[...][:, :1]
    z = dv * (aa_ref[...] + ab_ref[...] + g1_ref[...]) + b1_ref[...]
    r = jnp.maximum(z, 0.0)
    h = jnp.dot(r, w_ref[...], preferred_element_type=jnp.float32)
    g2_ref[...] = h * dv


_mm2 = pl.pallas_call(
    _mm2_body,
    grid=(N // BM,),
    in_specs=[
        pl.BlockSpec((BM, D), lambda i: (i, 0)),
        pl.BlockSpec((BM, D), lambda i: (i, 0)),
        pl.BlockSpec((BM, D), lambda i: (i, 0)),
        pl.BlockSpec((BM, 16), lambda i: (i, 0)),
        pl.BlockSpec((1, D), lambda i: (0, 0)),
        pl.BlockSpec((D, D), lambda i: (0, 0)),
    ],
    out_specs=pl.BlockSpec((BM, D), lambda i: (i, 0)),
    out_shape=jax.ShapeDtypeStruct((N, D), jnp.float32),
)


def _final_body(aa_ref, ab_ref, g2_ref, dv_ref, b2_ref, o_ref):
    dv = dv_ref[...][:, :1]
    o_ref[...] = dv * (aa_ref[...] + ab_ref[...] + g2_ref[...]) + b2_ref[...]


_final = pl.pallas_call(
    _final_body,
    grid=(N // BM,),
    in_specs=[
        pl.BlockSpec((BM, D), lambda i: (i, 0)),
        pl.BlockSpec((BM, D), lambda i: (i, 0)),
        pl.BlockSpec((BM, D), lambda i: (i, 0)),
        pl.BlockSpec((BM, 16), lambda i: (i, 0)),
        pl.BlockSpec((1, D), lambda i: (0, 0)),
    ],
    out_specs=pl.BlockSpec((BM, D), lambda i: (i, 0)),
    out_shape=jax.ShapeDtypeStruct((N, D), jnp.float32),
)


def kernel(x, edge_index, W1, b1, W2, b2):
    src = edge_index[0]
    dst = edge_index[1]
    pad = E_PAD - E
    srcp = jnp.concatenate([src, jnp.zeros((pad,), jnp.int32)])
    dstp = jnp.concatenate([dst, jnp.full((pad,), N, jnp.int32)])
    src3 = srcp.reshape(NW * CPT, CH)
    dst3 = dstp.reshape(NW * CPT, CH)

    z128 = jnp.zeros((RPT, D), jnp.float32)
    zdw = jnp.zeros((RPT, DW), jnp.float32)
    onesdw = jnp.ones((CH, DW), jnp.float32)

    degp = _deg_kernel(dst3, onesdw, zdw)
    g1, dinv = _mm1(x, W1, degp[0, :N, :16], degp[1, :N, :16])
    agg1 = _agg_kernel(g1, src3, dst3, z128)
    g2 = _mm2(agg1[0, :N], agg1[1, :N], g1, dinv,
              b1.reshape(1, D), W2)
    agg2 = _agg_kernel(g2, src3, dst3, z128)
    out = _final(agg2[0, :N], agg2[1, :N], g2, dinv,
                 b2.reshape(1, D))
    return out
